# Initial kernel scaffold; baseline (speedup 1.0000x reference)
import jax
import jax.numpy as jnp
from jax.experimental import pallas as pl


def kernel(x, edge_index, eps, W1, b1, gamma, beta, W2, b2):
    raise NotImplementedError("write your pallas kernel here")


# speedup vs baseline: 5.1018x; 5.1018x over previous
import functools

import jax
import jax.numpy as jnp
from jax import lax
from jax.experimental import pallas as pl
from jax.experimental.pallas import tpu as pltpu
from jax.experimental.pallas import tpu_sc as plsc

_NC = 2
_NS = 16


def _make_agg(N, D, E, k, Npad):
    nw = _NC * _NS
    epw = E // nw
    nchunks = epw // k
    rpt = Npad // _NS
    zrows = 128
    assert epw * nw == E and nchunks * k == epw and rpt * _NS == Npad
    assert rpt % zrows == 0 and k % 8 == 0 and epw % 8 == 0 and rpt % 8 == 0

    mesh = plsc.VectorSubcoreMesh(core_axis_name="c", subcore_axis_name="s")

    @functools.partial(
        pl.kernel,
        mesh=mesh,
        out_type=jax.ShapeDtypeStruct((_NC * Npad, D), jnp.float32),
        scratch_types=[
            pltpu.VMEM((k,), jnp.int32),
            pltpu.VMEM((k,), jnp.int32),
            pltpu.VMEM((k, D), jnp.float32),
            pltpu.VMEM((zrows, D), jnp.float32),
            pltpu.VMEM_SHARED((Npad, D), jnp.float32),
            pltpu.SemaphoreType.DMA,
        ],
    )
    def agg_kernel(x_hbm, src_hbm, dst_hbm, out_hbm,
                   src_v, dst_v, rows_v, zbuf, acc, sem):
        cid = lax.axis_index("c")
        sid = lax.axis_index("s")
        wid = cid * _NS + sid

        def zrow(i, c):
            for j in range(D // 16):
                zbuf[i, pl.ds(j * 16, 16)] = jnp.zeros((16,), jnp.float32)
            return c
        lax.fori_loop(0, zrows, zrow, 0)
        row0 = sid * rpt
        for t in range(rpt // zrows):
            pltpu.sync_copy(zbuf, acc.at[pl.ds(row0 + t * zrows, zrows)])
        plsc.subcore_barrier()

        ebase = wid * epw

        def chunk(j, c):
            off = ebase + j * k
            pltpu.sync_copy(src_hbm.at[pl.ds(off, k)], src_v)
            pltpu.sync_copy(dst_hbm.at[pl.ds(off, k)], dst_v)
            pltpu.async_copy(x_hbm.at[src_v], rows_v, sem).wait()
            pltpu.sync_copy(rows_v, acc.at[dst_v], add=True)
            return c
        lax.fori_loop(0, nchunks, chunk, 0)
        plsc.subcore_barrier()

        out0 = cid * Npad + row0
        for t in range(rpt // zrows):
            pltpu.sync_copy(acc.at[pl.ds(row0 + t * zrows, zrows)],
                            out_hbm.at[pl.ds(out0 + t * zrows, zrows)])

    return agg_kernel


def _mlp_body(eps_ref, x_ref, agg_ref, w1_ref, b1_ref, g_ref, be_ref,
              w2_ref, b2_ref, o_ref):
    n = x_ref.shape[0]
    npad = agg_ref.shape[0] // 2
    x = x_ref[...]
    agg = agg_ref[:n, :] + agg_ref[npad:npad + n, :]
    h = x * (1.0 + eps_ref[0, 0]) + agg
    y = jnp.dot(h, w1_ref[...], preferred_element_type=jnp.float32) + b1_ref[...]
    mean = jnp.mean(y, axis=0, keepdims=True)
    var = jnp.mean((y - mean) * (y - mean), axis=0, keepdims=True)
    z = (y - mean) * lax.rsqrt(var + 1e-5) * g_ref[...] + be_ref[...]
    z = jnp.maximum(z, 0.0)
    o = jnp.dot(z, w2_ref[...], preferred_element_type=jnp.float32) + b2_ref[...]
    o_ref[...] = x + jnp.maximum(o, 0.0)


def kernel(x, edge_index, eps, W1, b1, gamma, beta, W2, b2):
    N, D = x.shape
    E = edge_index.shape[1]

    src = edge_index[0]
    dst = edge_index[1]

    npad = ((N + _NS * 8 - 1) // (_NS * 8)) * (_NS * 8)
    npad = max(npad, _NS * 128)
    npad = ((npad + _NS * 128 - 1) // (_NS * 128)) * (_NS * 128)
    agg2 = _make_agg(N, D, E, 80, npad)(x, src, dst)

    vspec = pl.BlockSpec(memory_space=pltpu.VMEM)
    out = pl.pallas_call(
        _mlp_body,
        out_shape=jax.ShapeDtypeStruct((N, D), jnp.float32),
        in_specs=[pl.BlockSpec(memory_space=pltpu.SMEM)] + [vspec] * 8,
        out_specs=vspec,
    )(
        eps.reshape(1, 1),
        x,
        agg2,
        W1,
        b1.reshape(1, D),
        gamma.reshape(1, D),
        beta.reshape(1, D),
        W2,
        b2.reshape(1, D),
    )
    return out
